# baseline (speedup 1.0000x reference)
import functools

import jax
import jax.numpy as jnp
from jax import lax
from jax.experimental import pallas as pl
from jax.experimental.pallas import tpu as pltpu

T = 256
BN = 1024


def _gmm_body(eot_ref, x_ref, w_ref, b_ref, o_ref):
    acc = jnp.dot(x_ref[...], w_ref[0], preferred_element_type=jnp.float32)
    o_ref[...] = acc + b_ref[0]


def _grouped_matmul(x_pad, W, b, eot, P, D, E):
    MT = P // T
    NT = D // BN
    grid_spec = pltpu.PrefetchScalarGridSpec(
        num_scalar_prefetch=1,
        grid=(NT, MT),
        in_specs=[
            pl.BlockSpec((T, D), lambda n, m, eot: (m, 0)),
            pl.BlockSpec((1, D, BN), lambda n, m, eot: (eot[m], 0, n)),
            pl.BlockSpec((1, 1, BN), lambda n, m, eot: (eot[m], 0, n)),
        ],
        out_specs=pl.BlockSpec((T, BN), lambda n, m, eot: (m, n)),
    )
    return pl.pallas_call(
        _gmm_body,
        grid_spec=grid_spec,
        out_shape=jax.ShapeDtypeStruct((P, D), jnp.float32),
    )(eot, x_pad, W, b.reshape(E, 1, D))


def _route(e, E, P):
    N = e.shape[0]
    MT = P // T
    order = jnp.argsort(e, stable=True).astype(jnp.int32)
    se = e[order]
    counts = jnp.bincount(e, length=E + 1)[:E].astype(jnp.int32)
    padded = ((counts + T - 1) // T) * T
    goff = jnp.cumsum(padded) - padded
    coff = jnp.cumsum(counts) - counts
    r = jnp.arange(N, dtype=jnp.int32)
    se_c = jnp.minimum(se, E - 1)
    slot = goff[se_c] + (r - coff[se_c])
    svalid = se < E
    src = jnp.zeros((P,), jnp.int32).at[
        jnp.where(svalid, slot, P)].set(order, mode="drop")
    dest = jnp.zeros((N,), jnp.int32).at[order].set(
        jnp.where(svalid, slot, 0))
    tile_start = goff // T
    eot = (jnp.sum(jnp.arange(MT, dtype=jnp.int32)[:, None]
                   >= tile_start[None, :], axis=1) - 1)
    eot = jnp.clip(eot, 0, E - 1).astype(jnp.int32)
    return src, dest, e < E, eot


def kernel(points, region_ids, W, b):
    N, D = points.shape
    E = W.shape[0]
    P = ((N + E * (T - 1) + T - 1) // T) * T
    e = region_ids.astype(jnp.int32)
    src, dest, valid, eot = _route(e, E, P)
    x_pad = points[src]
    y_pad = _grouped_matmul(x_pad, W, b, eot, P, D, E)
    out = jnp.where(valid[:, None], y_pad[dest], 0.0)
    return out.reshape(-1, D)


# speedup vs baseline: 1.0666x; 1.0666x over previous
import functools

import jax
import jax.numpy as jnp
from jax import lax
from jax.experimental import pallas as pl
from jax.experimental.pallas import tpu as pltpu
from jax.experimental.pallas import tpu_sc as plsc

T = 256
BN = 1024
C = 16


def _sc_gather_rows(table, idx):
    V, D = table.shape
    R = idx.shape[0]
    info = plsc.get_sparse_core_info()
    NC, NS = info.num_cores, info.num_subcores
    NW = NC * NS
    rpw = R // NW
    nch = rpw // C
    mesh = plsc.VectorSubcoreMesh(core_axis_name="c", subcore_axis_name="s")

    @functools.partial(
        pl.kernel, mesh=mesh,
        out_type=jax.ShapeDtypeStruct((R, D), jnp.float32),
        scratch_types=[
            pltpu.VMEM((rpw,), jnp.int32),
            pltpu.VMEM((C, D), jnp.float32),
            pltpu.VMEM((C, D), jnp.float32),
            pltpu.SemaphoreType.DMA,
            pltpu.SemaphoreType.DMA,
            pltpu.SemaphoreType.DMA,
            pltpu.SemaphoreType.DMA,
        ],
    )
    def k(table_hbm, idx_hbm, out_hbm, idx_v, buf0, buf1, g0, g1, o0, o1):
        wid = lax.axis_index("s") * NC + lax.axis_index("c")
        base = wid * rpw
        pltpu.sync_copy(idx_hbm.at[pl.ds(base, rpw)], idx_v)
        bufs = (buf0, buf1)
        gsem = (g0, g1)
        osem = (o0, o1)
        gathers = [None, None]
        outs = [None, None]
        gathers[0] = pltpu.async_copy(
            table_hbm.at[idx_v.at[pl.ds(0, C)]], bufs[0], gsem[0])
        for c in range(nch):
            p = c % 2
            q = 1 - p
            if c + 1 < nch:
                if outs[q] is not None:
                    outs[q].wait()
                gathers[q] = pltpu.async_copy(
                    table_hbm.at[idx_v.at[pl.ds((c + 1) * C, C)]],
                    bufs[q], gsem[q])
            gathers[p].wait()
            outs[p] = pltpu.async_copy(
                bufs[p], out_hbm.at[pl.ds(base + c * C, C)], osem[p])
        outs[(nch - 1) % 2].wait()
        if nch > 1:
            outs[nch % 2].wait()

    return k(table, idx)


def _gmm_body(eot_ref, x_ref, w_ref, b_ref, o_ref):
    m = pl.program_id(1)
    mt = pl.num_programs(1) - 1

    @pl.when(m != mt)
    def _():
        acc = jnp.dot(x_ref[...], w_ref[0], preferred_element_type=jnp.float32)
        o_ref[...] = acc + b_ref[0]

    @pl.when(m == mt)
    def _():
        o_ref[...] = jnp.zeros_like(o_ref)


def _grouped_matmul(x_pad, W, b, eot, P, D, E):
    MT = P // T
    NT = D // BN
    grid_spec = pltpu.PrefetchScalarGridSpec(
        num_scalar_prefetch=1,
        grid=(NT, MT + 1),
        in_specs=[
            pl.BlockSpec((T, D), lambda n, m, eot: (jnp.minimum(m, MT - 1), 0)),
            pl.BlockSpec((1, D, BN), lambda n, m, eot: (eot[m], 0, n)),
            pl.BlockSpec((1, 1, BN), lambda n, m, eot: (eot[m], 0, n)),
        ],
        out_specs=pl.BlockSpec((T, BN), lambda n, m, eot: (m, n)),
    )
    return pl.pallas_call(
        _gmm_body,
        grid_spec=grid_spec,
        out_shape=jax.ShapeDtypeStruct((P + T, D), jnp.float32),
    )(eot, x_pad, W, b.reshape(E, 1, D))


def _route(e, E, P, MT):
    N = e.shape[0]
    order = jnp.argsort(e, stable=True).astype(jnp.int32)
    se = e[order]
    counts = jnp.bincount(e, length=E + 1)[:E].astype(jnp.int32)
    padded = ((counts + T - 1) // T) * T
    goff = jnp.cumsum(padded) - padded
    coff = jnp.cumsum(counts) - counts
    r = jnp.arange(N, dtype=jnp.int32)
    se_c = jnp.minimum(se, E - 1)
    slot = goff[se_c] + (r - coff[se_c])
    svalid = se < E
    src = jnp.zeros((P,), jnp.int32).at[
        jnp.where(svalid, slot, P)].set(order, mode="drop")
    dest = jnp.full((N,), P, jnp.int32).at[order].set(
        jnp.where(svalid, slot, P))
    tile_start = goff // T
    eot = (jnp.sum(jnp.arange(MT + 1, dtype=jnp.int32)[:, None]
                   >= tile_start[None, :], axis=1) - 1)
    eot = jnp.clip(eot, 0, E - 1).astype(jnp.int32)
    return src, dest, eot


def kernel(points, region_ids, W, b):
    N, D = points.shape
    E = W.shape[0]
    P = ((N + E * (T - 1) + T - 1) // T) * T
    MT = P // T
    e = region_ids.astype(jnp.int32)
    src, dest, eot = _route(e, E, P, MT)
    x_pad = _sc_gather_rows(points, src)
    y_ext = _grouped_matmul(x_pad, W, b, eot, P, D, E)
    out = _sc_gather_rows(y_ext, dest)
    return out.reshape(-1, D)
